# Initial kernel scaffold; baseline (speedup 1.0000x reference)
import jax
import jax.numpy as jnp
from jax.experimental import pallas as pl


def kernel(feat, label, centers):
    raise NotImplementedError("write your pallas kernel here")


# speedup vs baseline: 1.0844x; 1.0844x over previous
import functools

import jax
import jax.numpy as jnp
from jax import lax
from jax.experimental import pallas as pl
from jax.experimental.pallas import tpu as pltpu
from jax.experimental.pallas import tpu_sc as plsc

_FEAT_DIM = 128
_BATCH = 4096
_LAMBDA_C = 1.0
_LANES = 16

_info = plsc.get_sparse_core_info()
_NC, _NS = _info.num_cores, _info.num_subcores
_NW = _NC * _NS
_BPW = _BATCH // _NW


def _center_loss_partials(feat, label, centers):
  mesh = plsc.VectorSubcoreMesh(core_axis_name="c", subcore_axis_name="s")

  @functools.partial(
      pl.kernel,
      mesh=mesh,
      out_type=jax.ShapeDtypeStruct((_NW, _LANES), jnp.float32),
      scratch_types=[
          pltpu.VMEM((_BPW,), jnp.int32),
          pltpu.VMEM((_BPW, _FEAT_DIM), jnp.float32),
          pltpu.VMEM((_BPW, _FEAT_DIM), jnp.float32),
          pltpu.VMEM((_LANES,), jnp.float32),
          pltpu.SemaphoreType.DMA,
      ],
  )
  def k(feat_hbm, label_hbm, centers_hbm, out_hbm,
        idx_v, feat_v, rows_v, acc_v, sem):
    wid = lax.axis_index("s") * _NC + lax.axis_index("c")
    base = wid * _BPW
    pltpu.sync_copy(label_hbm.at[pl.ds(base, _BPW)], idx_v)
    gather = pltpu.async_copy(centers_hbm.at[idx_v], rows_v, sem)
    pltpu.sync_copy(feat_hbm.at[pl.ds(base, _BPW)], feat_v)
    gather.wait()

    def body(r, acc):
      for c in range(_FEAT_DIM // _LANES):
        d = feat_v[r, pl.ds(c * _LANES, _LANES)] - rows_v[r, pl.ds(c * _LANES, _LANES)]
        acc = acc + d * d
      return acc

    acc = lax.fori_loop(0, _BPW, body, jnp.zeros((_LANES,), jnp.float32))
    acc_v[...] = acc
    pltpu.sync_copy(acc_v, out_hbm.at[wid])

  return k(feat, label, centers)


def kernel(feat, label, centers):
  label = label.astype(jnp.int32)
  partials = _center_loss_partials(feat, label, centers)
  return _LAMBDA_C / 2.0 / _BATCH * jnp.sqrt(jnp.sum(partials))
